# baseline (speedup 1.0000x reference)
import functools

import jax
import jax.numpy as jnp
from jax import lax
from jax.experimental import pallas as pl
from jax.experimental.pallas import tpu as pltpu
from jax.experimental.pallas import tpu_sc as plsc

_NUM_EMBEDDINGS = 1000000
_DIM = 64
_BATCH = 16384
_HIST = 200
_B = _BATCH * _HIST

_NC = 2
_NS = 16
_NW = _NC * _NS

_BT = 128
_CT_PER_W = _BATCH // _BT // _NW
_IDS_PER_CT = _BT * _HIST


def _body(table_hbm, idx_hbm, out_hbm, idx_raw, idx_t, rows_v, trans_v,
          isem, gsem0, gsem1, wsem0, wsem1):
    wid = lax.axis_index("s") * _NC + lax.axis_index("c")
    gsems = (gsem0, gsem1)
    wsems = (wsem0, wsem1)

    iota = lax.iota(jnp.int32, 16)
    rowvecs = [iota + (c0 * 16) for c0 in range(8)]
    idxvecs = [(iota + (k0 * 16)) * _HIST for k0 in range(8)]

    def start_gather(h, s):
        pltpu.async_copy(table_hbm.at[idx_t.at[h]], rows_v.at[s], gsems[s])

    def wait_gather(s):
        pltpu.make_async_copy(table_hbm.at[idx_t.at[0]], rows_v.at[s],
                              gsems[s]).wait()

    def start_write(h, ct, s):
        pltpu.async_copy(trans_v.at[s], out_hbm.at[h, :, ct], wsems[s])

    def wait_write(s):
        pltpu.make_async_copy(trans_v.at[s], out_hbm.at[0, :, 0],
                              wsems[s]).wait()

    def transpose_strip(s):
        rows = rows_v.at[s]
        trans = trans_v.at[s]

        def rstep(R, carry):
            for r in range(8):
                d = R * 8 + r
                dvec = jnp.full((16,), d, jnp.int32)
                for c0 in range(8):
                    v = plsc.load_gather(rows, [rowvecs[c0], dvec])
                    trans[R, r, pl.ds(c0 * 16, 16)] = v
            return carry

        lax.fori_loop(0, 8, rstep, 0)

    def ct_step(cl, carry):
        ct = wid * _CT_PER_W + cl
        pltpu.async_copy(idx_hbm.at[pl.ds(ct * _IDS_PER_CT, _IDS_PER_CT)],
                         idx_raw, isem)
        pltpu.make_async_copy(idx_hbm.at[pl.ds(0, _IDS_PER_CT)], idx_raw,
                              isem).wait()

        def istep(h, carry):
            hvec = jnp.full((16,), h, jnp.int32)
            for k0 in range(8):
                v = plsc.load_gather(idx_raw, [idxvecs[k0] + hvec])
                idx_t[h, pl.ds(k0 * 16, 16)] = v
            return carry

        lax.fori_loop(0, _HIST, istep, 0)

        start_gather(0, 0)

        def hstep2(g, carry):
            for s in (0, 1):
                h = 2 * g + s

                @pl.when(h + 1 < _HIST)
                def _():
                    start_gather(h + 1, 1 - s)

                wait_gather(s)

                @pl.when(h > 1)
                def _():
                    wait_write(s)

                transpose_strip(s)
                start_write(h, ct, s)
            return carry

        lax.fori_loop(0, _HIST // 2, hstep2, 0)
        wait_write(0)
        wait_write(1)
        return carry

    lax.fori_loop(0, _CT_PER_W, ct_step, 0)


_gather = functools.partial(
    pl.kernel,
    out_type=jax.ShapeDtypeStruct((_HIST, 8, _BT, 8, _BT), jnp.float32),
    mesh=plsc.VectorSubcoreMesh(core_axis_name="c", subcore_axis_name="s"),
    scratch_types=[
        pltpu.VMEM((_IDS_PER_CT,), jnp.int32),
        pltpu.VMEM((_HIST, _BT), jnp.int32),
        pltpu.VMEM((2, _BT, _DIM), jnp.float32),
        pltpu.VMEM((2, 8, 8, _BT), jnp.float32),
        pltpu.SemaphoreType.DMA,
        pltpu.SemaphoreType.DMA,
        pltpu.SemaphoreType.DMA,
        pltpu.SemaphoreType.DMA,
        pltpu.SemaphoreType.DMA,
    ],
    compiler_params=pltpu.CompilerParams(use_tc_tiling_on_sc=False,
                                         needs_layout_passes=False),
)(_body)


@jax.jit
def kernel(token_ids, weights):
    flat = token_ids.reshape(_B)
    out5 = _gather(weights, flat)
    return out5.transpose(2, 4, 0, 1, 3).reshape(_BATCH, _HIST, _DIM)


# speedup vs baseline: 2.9482x; 2.9482x over previous
import functools

import jax
import jax.numpy as jnp
from jax import lax
from jax.experimental import pallas as pl
from jax.experimental.pallas import tpu as pltpu
from jax.experimental.pallas import tpu_sc as plsc

_NUM_EMBEDDINGS = 1000000
_DIM = 64
_BATCH = 16384
_HIST = 200
_B = _BATCH * _HIST

_NC = 2
_NS = 16
_NW = _NC * _NS

_B_PER_W = _B // _NW
_CHUNK = 800
_ROWS_PER_CHUNK = _CHUNK // _HIST
_BROWS_PER_W = _BATCH // _NW
_STEPS = _B_PER_W // _CHUNK
_G = _STEPS // 2


def _body(table_hbm, idx_hbm, out_hbm, idx_v, rows_v,
          isem0, isem1, gsem0, gsem1, osem0, osem1):
    wid = lax.axis_index("s") * _NC + lax.axis_index("c")
    base = wid * _B_PER_W
    isems = (isem0, isem1)
    gsems = (gsem0, gsem1)
    osems = (osem0, osem1)

    def start_idx(i, b):
        off = base + i * _CHUNK
        pltpu.async_copy(idx_hbm.at[pl.ds(off, _CHUNK)], idx_v.at[b], isems[b])

    def wait_idx(b):
        pltpu.make_async_copy(idx_hbm.at[pl.ds(0, _CHUNK)], idx_v.at[b],
                              isems[b]).wait()

    def start_gather(b):
        for j in range(_ROWS_PER_CHUNK):
            pltpu.async_copy(
                table_hbm.at[idx_v.at[b, pl.ds(j * _HIST, _HIST)]],
                rows_v.at[b, j], gsems[b])

    def wait_gather(b):
        for j in range(_ROWS_PER_CHUNK):
            pltpu.make_async_copy(
                table_hbm.at[idx_v.at[b, pl.ds(j * _HIST, _HIST)]],
                rows_v.at[b, j], gsems[b]).wait()

    def start_write(i, b):
        boff = wid * _BROWS_PER_W + i * _ROWS_PER_CHUNK
        pltpu.async_copy(
            rows_v.at[b],
            out_hbm.at[pl.ds(boff, _ROWS_PER_CHUNK), :, pl.ds(0, _DIM)],
            osems[b])

    def wait_write(b):
        pltpu.make_async_copy(
            rows_v.at[b],
            out_hbm.at[pl.ds(0, _ROWS_PER_CHUNK), :, pl.ds(0, _DIM)],
            osems[b]).wait()

    def chunk_step(i, b):
        nb = 1 - b

        @pl.when(i > 0)
        def _():
            wait_gather(nb)
            start_write(i - 1, nb)

        @pl.when(i + 1 < _STEPS)
        def _():
            start_idx(i + 1, nb)

        wait_idx(b)

        @pl.when(i > 1)
        def _():
            wait_write(b)

        start_gather(b)

    start_idx(0, 0)

    def gstep(g, carry):
        i0 = 2 * g
        chunk_step(i0, 0)
        chunk_step(i0 + 1, 1)
        return carry

    lax.fori_loop(0, _G, gstep, 0)

    last = _STEPS - 1
    wait_gather(1)
    start_write(last, 1)
    wait_write(0)
    wait_write(1)


_gather = functools.partial(
    pl.kernel,
    out_type=jax.ShapeDtypeStruct((_BATCH, _HIST, 2 * _DIM), jnp.float32),
    mesh=plsc.VectorSubcoreMesh(core_axis_name="c", subcore_axis_name="s"),
    scratch_types=[
        pltpu.VMEM((2, _CHUNK), jnp.int32),
        pltpu.VMEM((2, _ROWS_PER_CHUNK, _HIST, _DIM), jnp.float32),
        pltpu.SemaphoreType.DMA,
        pltpu.SemaphoreType.DMA,
        pltpu.SemaphoreType.DMA,
        pltpu.SemaphoreType.DMA,
        pltpu.SemaphoreType.DMA,
        pltpu.SemaphoreType.DMA,
    ],
    compiler_params=pltpu.CompilerParams(use_tc_tiling_on_sc=False),
)(_body)


@jax.jit
def kernel(token_ids, weights):
    flat = token_ids.reshape(_B)
    padded = _gather(weights, flat)
    return padded[:, :, :_DIM]


# speedup vs baseline: 2.9568x; 1.0029x over previous
import functools

import jax
import jax.numpy as jnp
from jax import lax
from jax.experimental import pallas as pl
from jax.experimental.pallas import tpu as pltpu
from jax.experimental.pallas import tpu_sc as plsc

_NUM_EMBEDDINGS = 1000000
_DIM = 64
_BATCH = 16384
_HIST = 200
_B = _BATCH * _HIST

_NC = 2
_NS = 16
_NW = _NC * _NS

_B_PER_W = _B // _NW
_CHUNK = 400
_ROWS_PER_CHUNK = _CHUNK // _HIST
_BROWS_PER_W = _BATCH // _NW
_STEPS = _B_PER_W // _CHUNK
_NBUF = 4
_G = _STEPS // _NBUF


def _body(table_hbm, idx_hbm, out_hbm, idx_v, rows_v,
          isem0, isem1, isem2, isem3,
          gsem0, gsem1, gsem2, gsem3,
          osem0, osem1, osem2, osem3):
    wid = lax.axis_index("s") * _NC + lax.axis_index("c")
    base = wid * _B_PER_W
    isems = (isem0, isem1, isem2, isem3)
    gsems = (gsem0, gsem1, gsem2, gsem3)
    osems = (osem0, osem1, osem2, osem3)

    def start_idx(i, b):
        off = base + i * _CHUNK
        pltpu.async_copy(idx_hbm.at[pl.ds(off, _CHUNK)], idx_v.at[b], isems[b])

    def wait_idx(b):
        pltpu.make_async_copy(idx_hbm.at[pl.ds(0, _CHUNK)], idx_v.at[b],
                              isems[b]).wait()

    def start_gather(b):
        for j in range(_ROWS_PER_CHUNK):
            pltpu.async_copy(
                table_hbm.at[idx_v.at[b, pl.ds(j * _HIST, _HIST)]],
                rows_v.at[b, j], gsems[b])

    def wait_gather(b):
        for j in range(_ROWS_PER_CHUNK):
            pltpu.make_async_copy(
                table_hbm.at[idx_v.at[b, pl.ds(j * _HIST, _HIST)]],
                rows_v.at[b, j], gsems[b]).wait()

    def start_write(i, b):
        boff = wid * _BROWS_PER_W + i * _ROWS_PER_CHUNK
        pltpu.async_copy(
            rows_v.at[b],
            out_hbm.at[pl.ds(boff, _ROWS_PER_CHUNK), :, pl.ds(0, _DIM)],
            osems[b])

    def wait_write(b):
        pltpu.make_async_copy(
            rows_v.at[b],
            out_hbm.at[pl.ds(0, _ROWS_PER_CHUNK), :, pl.ds(0, _DIM)],
            osems[b]).wait()

    def chunk_step(i, b):
        @pl.when(i > 1)
        def _():
            wait_gather((b - 2) % _NBUF)
            start_write(i - 2, (b - 2) % _NBUF)

        @pl.when(i + 2 < _STEPS)
        def _():
            start_idx(i + 2, (b + 2) % _NBUF)

        wait_idx(b)

        @pl.when(i > 3)
        def _():
            wait_write(b)

        start_gather(b)

    start_idx(0, 0)
    start_idx(1, 1)

    def gstep(g, carry):
        i0 = _NBUF * g
        for b in range(_NBUF):
            chunk_step(i0 + b, b)
        return carry

    lax.fori_loop(0, _G, gstep, 0)

    wait_gather((_STEPS - 2) % _NBUF)
    start_write(_STEPS - 2, (_STEPS - 2) % _NBUF)
    wait_gather((_STEPS - 1) % _NBUF)
    start_write(_STEPS - 1, (_STEPS - 1) % _NBUF)
    for b in range(_NBUF):
        wait_write(b)


_gather = functools.partial(
    pl.kernel,
    out_type=jax.ShapeDtypeStruct((_BATCH, _HIST, 2 * _DIM), jnp.float32),
    mesh=plsc.VectorSubcoreMesh(core_axis_name="c", subcore_axis_name="s"),
    scratch_types=[
        pltpu.VMEM((_NBUF, _CHUNK), jnp.int32),
        pltpu.VMEM((_NBUF, _ROWS_PER_CHUNK, _HIST, _DIM), jnp.float32),
    ] + [pltpu.SemaphoreType.DMA] * 12,
    compiler_params=pltpu.CompilerParams(use_tc_tiling_on_sc=False),
)(_body)


@jax.jit
def kernel(token_ids, weights):
    flat = token_ids.reshape(_B)
    padded = _gather(weights, flat)
    return padded[:, :, :_DIM]
